# baseline (speedup 1.0000x reference)
import functools

import jax
import jax.numpy as jnp
from jax import lax
from jax.experimental import pallas as pl
from jax.experimental.pallas import tpu as pltpu
from jax.experimental.pallas import tpu_sc as plsc

_BATCH = 4096
_SEQ = 50
_D = 64
_NW = 32
_BL = _BATCH // _NW
_NBUF = 5
_LOOKAHEAD = 3

_mesh = plsc.VectorSubcoreMesh(core_axis_name="c", subcore_axis_name="s")


@functools.partial(
    pl.kernel,
    mesh=_mesh,
    out_type=jax.ShapeDtypeStruct((_SEQ, _D // 8, _NW, 8, _BL), jnp.float32),
    scratch_types=[
        pltpu.VMEM((_SEQ, _BL), jnp.int32),
        pltpu.VMEM((_NBUF, _BL, _D), jnp.float32),
        pltpu.VMEM((_NBUF, _D // 8, 8, _BL), jnp.float32),
        pltpu.SemaphoreType.DMA((_NBUF,)),
        pltpu.SemaphoreType.DMA((_NBUF,)),
    ],
    compiler_params=pltpu.CompilerParams(
        use_tc_tiling_on_sc=False, needs_layout_passes=False),
)
def _embed_lookup(idx_hbm, table_hbm, out_hbm, idx_v, g_v, t_v, gsem, osem):
    wid = lax.axis_index("s") * 2 + lax.axis_index("c")
    pltpu.sync_copy(idx_hbm.at[wid], idx_v)
    for s in range(_LOOKAHEAD):
        pltpu.async_copy(table_hbm.at[idx_v.at[s]], g_v.at[s], gsem.at[s])

    lanes = lax.iota(jnp.int32, 16)
    cols = [lanes * 0 + d for d in range(_D)]

    def outer(g, _):
        for b in range(_NBUF):
            s = g * _NBUF + b
            nb = (b + _LOOKAHEAD) % _NBUF
            pltpu.make_async_copy(
                table_hbm.at[idx_v.at[b]], g_v.at[b], gsem.at[b]).wait()

            @pl.when(s >= _NBUF)
            def _():
                pltpu.make_async_copy(
                    t_v.at[b], out_hbm.at[0, :, wid], osem.at[b]).wait()

            gb = g_v.at[b]
            tb = t_v.at[b]

            def transpose_chunk(c, _):
                rows_c = lanes + c * 16
                for d in range(_D):
                    v = plsc.load_gather(gb, [rows_c, cols[d]])
                    tb[d // 8, d % 8, pl.ds(c * 16, 16)] = v
                return 0

            lax.fori_loop(0, _BL // 16, transpose_chunk, 0)
            pltpu.async_copy(tb, out_hbm.at[s, :, wid], osem.at[b])

            @pl.when(s + _LOOKAHEAD < _SEQ)
            def _():
                pltpu.async_copy(
                    table_hbm.at[idx_v.at[s + _LOOKAHEAD]], g_v.at[nb],
                    gsem.at[nb])

        return 0

    lax.fori_loop(0, _SEQ // _NBUF, outer, 0)
    for b in range(_NBUF):
        pltpu.make_async_copy(
            t_v.at[b], out_hbm.at[0, :, wid], osem.at[b]).wait()


def kernel(input_seq, embed_table):
    idx = input_seq.reshape(_NW, _BL, _SEQ).transpose(0, 2, 1)
    idx = idx.astype(jnp.int32)
    out5 = _embed_lookup(idx, embed_table)
    out = out5.transpose(2, 4, 0, 1, 3)
    return out.reshape(_BATCH, _SEQ, _D)


# speedup vs baseline: 1.4531x; 1.4531x over previous
import functools

import jax
import jax.numpy as jnp
from jax import lax
from jax.experimental import pallas as pl
from jax.experimental.pallas import tpu as pltpu
from jax.experimental.pallas import tpu_sc as plsc

_BATCH = 4096
_SEQ = 50
_D = 64
_NW = 32
_BL = _BATCH // _NW
_NBUF = 5
_LOOKAHEAD = 3

_mesh = plsc.VectorSubcoreMesh(core_axis_name="c", subcore_axis_name="s")


@functools.partial(
    pl.kernel,
    mesh=_mesh,
    out_type=jax.ShapeDtypeStruct((_SEQ, _D // 8, _NW, 8, _BL), jnp.float32),
    scratch_types=[
        pltpu.VMEM((_SEQ, _BL), jnp.int32),
        pltpu.VMEM((_NBUF, _BL, _D), jnp.float32),
        pltpu.VMEM((_NBUF, _D // 8, 8, _BL), jnp.float32),
        pltpu.SemaphoreType.DMA((_NBUF,)),
        pltpu.SemaphoreType.DMA((_NBUF,)),
    ],
    compiler_params=pltpu.CompilerParams(
        use_tc_tiling_on_sc=False, needs_layout_passes=False),
)
def _embed_lookup(idx_hbm, table_hbm, out_hbm, idx_v, g_v, t_v, gsem, osem):
    wid = lax.axis_index("s") * 2 + lax.axis_index("c")
    pltpu.sync_copy(idx_hbm.at[wid], idx_v)
    for s in range(_LOOKAHEAD):
        pltpu.async_copy(table_hbm.at[idx_v.at[s]], g_v.at[s], gsem.at[s])

    lanes = lax.iota(jnp.int32, 16)
    cols = [lanes * 0 + d for d in range(_D)]

    def outer(g, _):
        for b in range(_NBUF):
            s = g * _NBUF + b
            nb = (b + _LOOKAHEAD) % _NBUF
            pltpu.make_async_copy(
                table_hbm.at[idx_v.at[b]], g_v.at[b], gsem.at[b]).wait()

            @pl.when(s >= _NBUF)
            def _():
                pltpu.make_async_copy(
                    t_v.at[b], out_hbm.at[0, :, wid], osem.at[b]).wait()

            gb = g_v.at[b]
            tb = t_v.at[b]

            @plsc.parallel_loop(0, _BL // 16)
            def _(c):
                rows_c = lanes + c * 16
                for d in range(_D):
                    v = plsc.load_gather(gb, [rows_c, cols[d]])
                    tb[d // 8, d % 8, pl.ds(c * 16, 16)] = v
            pltpu.async_copy(tb, out_hbm.at[s, :, wid], osem.at[b])

            @pl.when(s + _LOOKAHEAD < _SEQ)
            def _():
                pltpu.async_copy(
                    table_hbm.at[idx_v.at[s + _LOOKAHEAD]], g_v.at[nb],
                    gsem.at[nb])

        return 0

    lax.fori_loop(0, _SEQ // _NBUF, outer, 0)
    for b in range(_NBUF):
        pltpu.make_async_copy(
            t_v.at[b], out_hbm.at[0, :, wid], osem.at[b]).wait()


def kernel(input_seq, embed_table):
    idx = input_seq.reshape(_NW, _BL, _SEQ).transpose(0, 2, 1)
    idx = idx.astype(jnp.int32)
    out5 = _embed_lookup(idx, embed_table)
    out = out5.transpose(2, 4, 0, 1, 3)
    return out.reshape(_BATCH, _SEQ, _D)


# speedup vs baseline: 3.2839x; 2.2599x over previous
import functools

import jax
import jax.numpy as jnp
from jax import lax
from jax.experimental import pallas as pl
from jax.experimental.pallas import tpu as pltpu
from jax.experimental.pallas import tpu_sc as plsc

_BATCH = 4096
_SEQ = 50
_D = 64
_NW = 32
_BL = _BATCH // _NW
_NBUF = 5
_LOOKAHEAD = 3
_TP = 129

_mesh = plsc.VectorSubcoreMesh(core_axis_name="c", subcore_axis_name="s")


@functools.partial(
    pl.kernel,
    mesh=_mesh,
    out_type=jax.ShapeDtypeStruct((_SEQ, _D // 8, _NW, 8, _BL), jnp.float32),
    scratch_types=[
        pltpu.VMEM((_SEQ, _BL), jnp.int32),
        pltpu.VMEM((_NBUF, _BL, _D), jnp.float32),
        pltpu.VMEM((_NBUF, _D // 8, 8, _TP), jnp.float32),
        pltpu.SemaphoreType.DMA((_NBUF,)),
        pltpu.SemaphoreType.DMA((_NBUF,)),
    ],
    compiler_params=pltpu.CompilerParams(
        use_tc_tiling_on_sc=False, needs_layout_passes=False),
)
def _embed_lookup(idx_hbm, table_hbm, out_hbm, idx_v, g_v, t_v, gsem, osem):
    wid = lax.axis_index("s") * 2 + lax.axis_index("c")
    pltpu.sync_copy(idx_hbm.at[wid], idx_v)
    for s in range(_LOOKAHEAD):
        pltpu.async_copy(table_hbm.at[idx_v.at[s]], g_v.at[s], gsem.at[s])

    lanes = lax.iota(jnp.int32, 16)
    dt_vecs = [(lanes + 16 * q) // 8 for q in range(_D // 16)]
    dr_vecs = [(lanes + 16 * q) % 8 for q in range(_D // 16)]

    def outer(g, _):
        for b in range(_NBUF):
            s = g * _NBUF + b
            nb = (b + _LOOKAHEAD) % _NBUF
            pltpu.make_async_copy(
                table_hbm.at[idx_v.at[b]], g_v.at[b], gsem.at[b]).wait()

            @pl.when(s >= _NBUF)
            def _():
                pltpu.make_async_copy(
                    t_v.at[b, :, :, pl.ds(0, _BL)],
                    out_hbm.at[0, :, wid], osem.at[b]).wait()

            gb = g_v.at[b]
            tb = t_v.at[b]

            @plsc.parallel_loop(0, _BL, unroll=4)
            def _(r):
                rsplat = lanes * 0 + r
                for q in range(_D // 16):
                    v = gb[r, pl.ds(16 * q, 16)]
                    plsc.store_scatter(
                        tb, [dt_vecs[q], dr_vecs[q], rsplat], v)

            pltpu.async_copy(
                t_v.at[b, :, :, pl.ds(0, _BL)], out_hbm.at[s, :, wid],
                osem.at[b])

            @pl.when(s + _LOOKAHEAD < _SEQ)
            def _():
                pltpu.async_copy(
                    table_hbm.at[idx_v.at[s + _LOOKAHEAD]], g_v.at[nb],
                    gsem.at[nb])

        return 0

    lax.fori_loop(0, _SEQ // _NBUF, outer, 0)
    for b in range(_NBUF):
        pltpu.make_async_copy(
            t_v.at[b, :, :, pl.ds(0, _BL)], out_hbm.at[0, :, wid],
            osem.at[b]).wait()


def kernel(input_seq, embed_table):
    idx = input_seq.reshape(_NW, _BL, _SEQ).transpose(0, 2, 1)
    idx = idx.astype(jnp.int32)
    out5 = _embed_lookup(idx, embed_table)
    out = out5.transpose(2, 4, 0, 1, 3)
    return out.reshape(_BATCH, _SEQ, _D)
